# baseline (speedup 1.0000x reference)
import functools

import jax
import jax.numpy as jnp
from jax import lax
from jax.experimental import pallas as pl
from jax.experimental.pallas import tpu as pltpu
from jax.experimental.pallas import tpu_sc as plsc

NC = 2
NS = 16
NW = NC * NS
LANES = 16
CHUNK = 128


@functools.lru_cache(maxsize=None)
def _build(vocab: int, batch: int, dim: int):
    assert dim == LANES
    b_per_w = batch // NW
    n_chunks = b_per_w // CHUNK
    assert b_per_w % CHUNK == 0

    mesh = plsc.VectorSubcoreMesh(core_axis_name="c", subcore_axis_name="s")

    @functools.partial(
        pl.kernel,
        out_type=jax.ShapeDtypeStruct((batch,), jnp.float32),
        mesh=mesh,
        scratch_types=[
            pltpu.VMEM((n_chunks, CHUNK), jnp.int32),
            pltpu.VMEM((b_per_w, dim), jnp.float32),
            pltpu.VMEM((b_per_w,), jnp.float32),
            pltpu.VMEM((dim,), jnp.float32),
            pltpu.VMEM((LANES,), jnp.float32),
            pltpu.SemaphoreType.DMA,
        ],
        compiler_params=pltpu.CompilerParams(
            needs_layout_passes=False, use_tc_tiling_on_sc=False),
    )
    def sc_kernel(idx_hbm, table_hbm, w_hbm, b_hbm, out_hbm,
                  idx_v, rows_v, out_v, w_v, b_v, sem):
        wid = lax.axis_index("s") * NC + lax.axis_index("c")
        base = wid * b_per_w

        pltpu.sync_copy(idx_hbm.at[wid], idx_v)
        pltpu.sync_copy(w_hbm, w_v)
        pltpu.sync_copy(b_hbm, b_v)

        copies = [
            pltpu.async_copy(
                table_hbm.at[idx_v.at[k]],
                rows_v.at[pl.ds(k * CHUNK, CHUNK)],
                sem,
            )
            for k in range(n_chunks)
        ]
        for c in copies:
            c.wait()

        w = w_v[...]
        b = b_v[...]
        lane = lax.iota(jnp.int32, LANES)

        def body(blk, carry):
            r0 = blk * LANES
            row_idx = r0 + lane
            acc = b
            for j in range(dim):
                col = plsc.load_gather(
                    rows_v, [row_idx, jnp.full((LANES,), j, jnp.int32)])
                acc = acc + col * w[j]
            out_v[pl.ds(r0, LANES)] = acc
            return carry

        lax.fori_loop(0, b_per_w // LANES, body, 0)

        pltpu.sync_copy(out_v, out_hbm.at[pl.ds(base, b_per_w)])

    return sc_kernel


def kernel(x, table, fc_w, fc_b):
    batch = x.shape[0]
    vocab, dim = table.shape
    idx = x.astype(jnp.int32).reshape(NW, batch // NW // CHUNK, CHUNK)
    w = fc_w.reshape(dim).astype(jnp.float32)
    b = jnp.broadcast_to(fc_b.astype(jnp.float32), (LANES,))
    out = _build(vocab, batch, dim)(idx, table, w, b)
    return out.reshape(batch, 1)


# speedup vs baseline: 9.2328x; 9.2328x over previous
import functools

import jax
import jax.numpy as jnp
from jax import lax
from jax.experimental import pallas as pl
from jax.experimental.pallas import tpu as pltpu
from jax.experimental.pallas import tpu_sc as plsc

NC = 2
NS = 16
NW = NC * NS
CHUNK = 128
COLS = 65536


@functools.lru_cache(maxsize=None)
def _build_matvec(vocab: int, dim: int):
    grid = (vocab + COLS - 1) // COLS

    def body(t_ref, w_ref, b_ref, y_ref):
        t = t_ref[...]
        w = w_ref[...]
        y_ref[...] = jnp.sum(t * w, axis=0) + b_ref[0]

    return pl.pallas_call(
        body,
        grid=(grid,),
        in_specs=[
            pl.BlockSpec((dim, COLS), lambda i: (0, i)),
            pl.BlockSpec((dim, 1), lambda i: (0, 0)),
            pl.BlockSpec(memory_space=pltpu.SMEM),
        ],
        out_specs=pl.BlockSpec((COLS,), lambda i: (i,)),
        out_shape=jax.ShapeDtypeStruct((vocab,), jnp.float32),
        compiler_params=pltpu.CompilerParams(
            dimension_semantics=("arbitrary",)),
    )


@functools.lru_cache(maxsize=None)
def _build_gather(vocab: int, batch: int):
    b_per_w = batch // NW
    n_chunks = b_per_w // CHUNK
    assert b_per_w % CHUNK == 0

    mesh = plsc.VectorSubcoreMesh(core_axis_name="c", subcore_axis_name="s")

    @functools.partial(
        pl.kernel,
        out_type=jax.ShapeDtypeStruct((batch,), jnp.float32),
        mesh=mesh,
        scratch_types=[
            pltpu.VMEM((n_chunks, CHUNK), jnp.int32),
            pltpu.VMEM((b_per_w,), jnp.float32),
            pltpu.SemaphoreType.DMA,
        ],
        compiler_params=pltpu.CompilerParams(
            needs_layout_passes=False, use_tc_tiling_on_sc=False),
    )
    def sc_kernel(idx_hbm, y_hbm, out_hbm, idx_v, out_v, sem):
        wid = lax.axis_index("s") * NC + lax.axis_index("c")
        base = wid * b_per_w

        pltpu.sync_copy(idx_hbm.at[wid], idx_v)

        copies = [
            pltpu.async_copy(
                y_hbm.at[idx_v.at[k]],
                out_v.at[pl.ds(k * CHUNK, CHUNK)],
                sem,
            )
            for k in range(n_chunks)
        ]
        for c in copies:
            c.wait()

        pltpu.sync_copy(out_v, out_hbm.at[pl.ds(base, b_per_w)])

    return sc_kernel


def kernel(x, table, fc_w, fc_b):
    batch = x.shape[0]
    vocab, dim = table.shape
    table_t = table.T
    w = fc_w.reshape(dim, 1).astype(jnp.float32)
    y = _build_matvec(vocab, dim)(table_t, w, fc_b.astype(jnp.float32))
    idx = x.astype(jnp.int32).reshape(NW, batch // NW // CHUNK, CHUNK)
    out = _build_gather(vocab, batch)(idx, y)
    return out.reshape(batch, 1)


# speedup vs baseline: 9.8990x; 1.0722x over previous
import functools

import jax
import jax.numpy as jnp
from jax import lax
from jax.experimental import pallas as pl
from jax.experimental.pallas import tpu as pltpu
from jax.experimental.pallas import tpu_sc as plsc

NC = 2
NS = 16
NW = NC * NS
CHUNK = 128
COLS = 131072


@functools.lru_cache(maxsize=None)
def _build_matvec(vocab: int, dim: int):
    grid = (vocab + COLS - 1) // COLS

    def body(t_ref, w_ref, b_ref, y_ref):
        t = t_ref[...]
        w = w_ref[...]
        y_ref[...] = jnp.sum(t * w, axis=0) + b_ref[0]

    return pl.pallas_call(
        body,
        grid=(grid,),
        in_specs=[
            pl.BlockSpec((dim, COLS), lambda i: (0, i)),
            pl.BlockSpec((dim, 1), lambda i: (0, 0)),
            pl.BlockSpec(memory_space=pltpu.SMEM),
        ],
        out_specs=pl.BlockSpec((COLS,), lambda i: (i,)),
        out_shape=jax.ShapeDtypeStruct((vocab,), jnp.float32),
        compiler_params=pltpu.CompilerParams(
            dimension_semantics=("arbitrary",)),
    )


@functools.lru_cache(maxsize=None)
def _build_gather(vocab: int, batch: int):
    b_per_w = batch // NW
    n_chunks = b_per_w // CHUNK
    assert b_per_w % CHUNK == 0

    mesh = plsc.VectorSubcoreMesh(core_axis_name="c", subcore_axis_name="s")

    @functools.partial(
        pl.kernel,
        out_type=jax.ShapeDtypeStruct((batch,), jnp.float32),
        mesh=mesh,
        scratch_types=[
            pltpu.VMEM((n_chunks, CHUNK), jnp.int32),
            pltpu.VMEM((b_per_w,), jnp.float32),
            pltpu.SemaphoreType.DMA,
        ],
        compiler_params=pltpu.CompilerParams(
            needs_layout_passes=False, use_tc_tiling_on_sc=False),
    )
    def sc_kernel(idx_hbm, y_hbm, out_hbm, idx_v, out_v, sem):
        wid = lax.axis_index("s") * NC + lax.axis_index("c")
        base = wid * b_per_w

        pltpu.sync_copy(idx_hbm.at[wid], idx_v)

        copies = [
            pltpu.async_copy(
                y_hbm.at[idx_v.at[k]],
                out_v.at[pl.ds(k * CHUNK, CHUNK)],
                sem,
            )
            for k in range(n_chunks)
        ]
        for c in copies:
            c.wait()

        pltpu.sync_copy(out_v, out_hbm.at[pl.ds(base, b_per_w)])

    return sc_kernel


def kernel(x, table, fc_w, fc_b):
    batch = x.shape[0]
    vocab, dim = table.shape
    table_t = table.T
    w = fc_w.reshape(dim, 1).astype(jnp.float32)
    y = _build_matvec(vocab, dim)(table_t, w, fc_b.astype(jnp.float32))
    idx = x.astype(jnp.int32).reshape(NW, batch // NW // CHUNK, CHUNK)
    out = _build_gather(vocab, batch)(idx, y)
    return out.reshape(batch, 1)
